# Initial kernel scaffold; baseline (speedup 1.0000x reference)
import jax
import jax.numpy as jnp
from jax.experimental import pallas as pl


def kernel(A_dense, O, rot_rows, rot_cols, wav_idx, act_idx):
    raise NotImplementedError("write your pallas kernel here")


# speedup vs baseline: 3.0817x; 3.0817x over previous
import jax
import jax.numpy as jnp
from jax.experimental import pallas as pl

N = 1024
K = 128
NB = N - K
NA = N - 8
KA = K - 8


def _strip_kernel(a_top_ref, a_colsL_ref, o_ref,
                  b_ref, dtop_ref, artop_ref, dleft_ref, arleft_ref,
                  fctl_ref, fctr_ref, fcleft_ref, fwtl_ref, mc_ref, mw_ref):
    f32 = jnp.float32
    a_top = a_top_ref[...]
    a_left = a_colsL_ref[...][K:, :]
    o = o_ref[...]

    o128 = o.reshape(K, 16)
    x = jnp.concatenate([o128] * 8, axis=1)
    rr = jax.lax.broadcasted_iota(jnp.int32, (K, K), 0)
    cc = jax.lax.broadcasted_iota(jnp.int32, (K, K), 1)
    b = jnp.where((rr // 16) == (cc // 16), x, 0.0)
    b_ref[...] = b

    gp = jax.lax.broadcasted_iota(jnp.int32, (KA, K), 0)
    gq = jax.lax.broadcasted_iota(jnp.int32, (KA, K), 1)
    g = ((gp // 15) * 16 + (gp % 15) + 1 == gq).astype(f32)
    ep = jax.lax.broadcasted_iota(jnp.int32, (8, K), 0)
    eq = jax.lax.broadcasted_iota(jnp.int32, (8, K), 1)
    e = (ep * 16 == eq).astype(f32)

    dot = lambda u, v: jnp.dot(u, v, preferred_element_type=f32)

    t = dot(b, a_top)
    m = dot(t[:, :K], b.T)
    t896 = dot(a_left, b.T)

    ri = jax.lax.broadcasted_iota(jnp.int32, (K, N), 0)
    ci = jax.lax.broadcasted_iota(jnp.int32, (K, N), 1)
    act_r = (ri % 16) != 0
    act_c = (ci >= K) | ((ci % 16) != 0)
    af_top = jnp.concatenate([m, t[:, K:]], axis=1)
    d_top = jnp.where((ri == ci) | (act_r & act_c), af_top, 0.0)
    dtop_ref[...] = d_top

    colmask = ((jax.lax.broadcasted_iota(jnp.int32, (NB, K), 1) % 16) != 0)
    d_left = jnp.where(colmask, t896, 0.0)
    dleft_ref[...] = d_left

    s = dot(b.T, d_top)
    artop_ref[...] = jnp.concatenate([dot(s[:, :K], b), s[:, K:]], axis=1)
    arleft_ref[...] = dot(d_left, b)

    fctl_ref[...] = dot(dot(g, m), g.T)
    fctr_ref[...] = dot(g, t[:, K:])
    fcleft_ref[...] = dot(t896, g.T)

    fwtl_ref[...] = dot(g, b)
    eme = dot(dot(e, m), e.T)
    i8 = jax.lax.broadcasted_iota(jnp.int32, (8, 8), 0)
    j8 = jax.lax.broadcasted_iota(jnp.int32, (8, 8), 1)
    mc_ref[...] = jnp.where(i8 == j8, eme, 0.0)
    mw_ref[...] = jnp.concatenate(
        [dot(e, b), jnp.zeros((8, NB), f32)], axis=1)


def _assemble_big_kernel(a_ref, b_ref, dtop_ref, artop_ref, dleft_ref,
                         arleft_ref, d_ref, ar_ref, right_ref):
    i = pl.program_id(0)

    @pl.when(i == 0)
    def _():
        d_ref[...] = dtop_ref[...]
        ar_ref[...] = artop_ref[...]
        right_ref[...] = jnp.concatenate(
            [b_ref[...], jnp.zeros((K, N - K), jnp.float32)], axis=1)

    @pl.when(i > 0)
    def _():
        bottom = a_ref[...][:, K:]
        d_ref[...] = jnp.concatenate([dleft_ref[...], bottom], axis=1)
        ar_ref[...] = jnp.concatenate([arleft_ref[...], bottom], axis=1)
        r = K * i + jax.lax.broadcasted_iota(jnp.int32, (K, N), 0)
        c = jax.lax.broadcasted_iota(jnp.int32, (K, N), 1)
        right_ref[...] = (r == c).astype(jnp.float32)


def _assemble_father_kernel(a_ref, fctl_ref, fctr_ref, fcleft_ref, fwtl_ref,
                            fc_ref, fw_ref):
    i = pl.program_id(0)

    @pl.when(i < 15)
    def _():
        fc_ref[...] = jnp.concatenate([fctl_ref[...], fctr_ref[...]], axis=1)
        fw_ref[...] = jnp.concatenate(
            [fwtl_ref[...], jnp.zeros((8, NB), jnp.float32)], axis=1)

    @pl.when(i >= 15)
    def _():
        fc_ref[...] = jnp.concatenate(
            [fcleft_ref[...], a_ref[...][:, K:]], axis=1)
        r = 8 * i + jax.lax.broadcasted_iota(jnp.int32, (8, N), 0)
        c = jax.lax.broadcasted_iota(jnp.int32, (8, N), 1)
        fw_ref[...] = (c == r + 8).astype(jnp.float32)


def kernel(A_dense, O, rot_rows, rot_cols, wav_idx, act_idx):
    f32 = jnp.float32
    sds = jax.ShapeDtypeStruct

    strips = pl.pallas_call(
        _strip_kernel,
        grid=(1,),
        in_specs=[
            pl.BlockSpec((K, N), lambda i: (0, 0)),
            pl.BlockSpec((N, K), lambda i: (0, 0)),
            pl.BlockSpec((8, 16, 16), lambda i: (0, 0, 0)),
        ],
        out_specs=[
            pl.BlockSpec((K, K), lambda i: (0, 0)),
            pl.BlockSpec((K, N), lambda i: (0, 0)),
            pl.BlockSpec((K, N), lambda i: (0, 0)),
            pl.BlockSpec((NB, K), lambda i: (0, 0)),
            pl.BlockSpec((NB, K), lambda i: (0, 0)),
            pl.BlockSpec((KA, KA), lambda i: (0, 0)),
            pl.BlockSpec((KA, NB), lambda i: (0, 0)),
            pl.BlockSpec((NB, KA), lambda i: (0, 0)),
            pl.BlockSpec((KA, K), lambda i: (0, 0)),
            pl.BlockSpec((8, 8), lambda i: (0, 0)),
            pl.BlockSpec((8, N), lambda i: (0, 0)),
        ],
        out_shape=[
            sds((K, K), f32),
            sds((K, N), f32),
            sds((K, N), f32),
            sds((NB, K), f32),
            sds((NB, K), f32),
            sds((KA, KA), f32),
            sds((KA, NB), f32),
            sds((NB, KA), f32),
            sds((KA, K), f32),
            sds((8, 8), f32),
            sds((8, N), f32),
        ],
    )(A_dense, A_dense, O)
    (b, d_top, ar_top, d_left, ar_left,
     fctl, fctr, fcleft, fwtl, mc, mw) = strips

    d, a_rec, right = pl.pallas_call(
        _assemble_big_kernel,
        grid=(8,),
        in_specs=[
            pl.BlockSpec((K, N), lambda i: (i, 0)),
            pl.BlockSpec((K, K), lambda i: (0, 0)),
            pl.BlockSpec((K, N), lambda i: (0, 0)),
            pl.BlockSpec((K, N), lambda i: (0, 0)),
            pl.BlockSpec((K, K), lambda i: (jnp.maximum(i - 1, 0), 0)),
            pl.BlockSpec((K, K), lambda i: (jnp.maximum(i - 1, 0), 0)),
        ],
        out_specs=[
            pl.BlockSpec((K, N), lambda i: (i, 0)),
            pl.BlockSpec((K, N), lambda i: (i, 0)),
            pl.BlockSpec((K, N), lambda i: (i, 0)),
        ],
        out_shape=[sds((N, N), f32)] * 3,
    )(A_dense, b, d_top, ar_top, d_left, ar_left)

    fc, fw = pl.pallas_call(
        _assemble_father_kernel,
        grid=(127,),
        in_specs=[
            pl.BlockSpec((8, N), lambda i: (jnp.minimum(i + 1, 127), 0)),
            pl.BlockSpec((8, KA), lambda i: (jnp.minimum(i, 14), 0)),
            pl.BlockSpec((8, NB), lambda i: (jnp.minimum(i, 14), 0)),
            pl.BlockSpec((8, KA), lambda i: (jnp.maximum(i - 15, 0), 0)),
            pl.BlockSpec((8, K), lambda i: (jnp.minimum(i, 14), 0)),
        ],
        out_specs=[
            pl.BlockSpec((8, NA), lambda i: (i, 0)),
            pl.BlockSpec((8, N), lambda i: (i, 0)),
        ],
        out_shape=[sds((NA, NA), f32), sds((NA, N), f32)],
    )(A_dense, fctl, fctr, fcleft, fwtl)

    return (a_rec, right, d, mc, fc, mw, fw)


# speedup vs baseline: 15.1511x; 4.9165x over previous
import jax
import jax.numpy as jnp
from jax.experimental import pallas as pl

N = 1024
K = 128
NB = N - K
NA = N - 8
KA = K - 8


def _strip_kernel(a_top_ref, a_colsL_ref, o_ref,
                  b_ref, dtop_ref, artop_ref, dleft_ref, arleft_ref,
                  fctl_ref, fctr_ref, fcleft_ref, fwtl_ref, mc_ref, mw_ref):
    f32 = jnp.float32
    a_top = a_top_ref[...]
    a_left = a_colsL_ref[...][K:, :]
    o = o_ref[...]

    o128 = o.reshape(K, 16)
    x = jnp.concatenate([o128] * 8, axis=1)
    rr = jax.lax.broadcasted_iota(jnp.int32, (K, K), 0)
    cc = jax.lax.broadcasted_iota(jnp.int32, (K, K), 1)
    b = jnp.where((rr // 16) == (cc // 16), x, 0.0)
    b_ref[...] = b

    gp = jax.lax.broadcasted_iota(jnp.int32, (KA, K), 0)
    gq = jax.lax.broadcasted_iota(jnp.int32, (KA, K), 1)
    g = ((gp // 15) * 16 + (gp % 15) + 1 == gq).astype(f32)
    ep = jax.lax.broadcasted_iota(jnp.int32, (8, K), 0)
    eq = jax.lax.broadcasted_iota(jnp.int32, (8, K), 1)
    e = (ep * 16 == eq).astype(f32)

    dot = lambda u, v: jnp.dot(u, v, preferred_element_type=f32)

    t = dot(b, a_top)
    m = dot(t[:, :K], b.T)
    t896 = dot(a_left, b.T)

    ri = jax.lax.broadcasted_iota(jnp.int32, (K, N), 0)
    ci = jax.lax.broadcasted_iota(jnp.int32, (K, N), 1)
    act_r = (ri % 16) != 0
    act_c = (ci >= K) | ((ci % 16) != 0)
    af_top = jnp.concatenate([m, t[:, K:]], axis=1)
    d_top = jnp.where((ri == ci) | (act_r & act_c), af_top, 0.0)
    dtop_ref[...] = d_top

    colmask = ((jax.lax.broadcasted_iota(jnp.int32, (NB, K), 1) % 16) != 0)
    d_left = jnp.where(colmask, t896, 0.0)
    dleft_ref[...] = d_left

    s = dot(b.T, d_top)
    artop_ref[...] = jnp.concatenate([dot(s[:, :K], b), s[:, K:]], axis=1)
    arleft_ref[...] = dot(d_left, b)

    fctl_ref[...] = dot(dot(g, m), g.T)
    fctr_ref[...] = dot(g, t[:, K:])
    fcleft_ref[...] = dot(t896, g.T)

    fwtl_ref[...] = dot(g, b)
    eme = dot(dot(e, m), e.T)
    i8 = jax.lax.broadcasted_iota(jnp.int32, (8, 8), 0)
    j8 = jax.lax.broadcasted_iota(jnp.int32, (8, 8), 1)
    mc_ref[...] = jnp.where(i8 == j8, eme, 0.0)
    mw_ref[...] = jnp.concatenate(
        [dot(e, b), jnp.zeros((8, NB), f32)], axis=1)


def _assemble_kernel(a_ref, apeek_ref, b_ref, dtop_ref, artop_ref,
                     dleft_ref, arleft_ref, fctl_ref, fctr_ref,
                     fcleft_ref, fcpeek_ref, fwtl_ref,
                     d_ref, ar_ref, right_ref, fc_ref, fw_ref):
    i = pl.program_id(0)
    f32 = jnp.float32
    c1024 = jax.lax.broadcasted_iota(jnp.int32, (K, N), 1)
    r1024 = jax.lax.broadcasted_iota(jnp.int32, (K, N), 0)

    @pl.when(i == 0)
    def _():
        d_ref[...] = dtop_ref[...]
        ar_ref[...] = artop_ref[...]
        right_ref[...] = jnp.concatenate(
            [b_ref[...], jnp.zeros((K, NB), f32)], axis=1)
        fc_top = jnp.concatenate([fctl_ref[...], fctr_ref[...]], axis=1)
        fc_bot = jnp.concatenate(
            [fcpeek_ref[...], apeek_ref[...][:, K:]], axis=1)
        fc_ref[...] = jnp.concatenate([fc_top, fc_bot], axis=0)
        fw_top = jnp.concatenate(
            [fwtl_ref[...], jnp.zeros((KA, NB), f32)], axis=1)
        fw_bot = (c1024[:8] == r1024[:8] + 128).astype(f32)
        fw_ref[...] = jnp.concatenate([fw_top, fw_bot], axis=0)

    @pl.when(i > 0)
    def _():
        bottom = a_ref[...][:, K:]
        d_ref[...] = jnp.concatenate([dleft_ref[...], bottom], axis=1)
        ar_ref[...] = jnp.concatenate([arleft_ref[...], bottom], axis=1)
        right_ref[...] = (K * i + r1024 == c1024).astype(f32)
        fcleft_win = jnp.concatenate(
            [fcleft_ref[...][8:], fcpeek_ref[...]], axis=0)
        a_win = jnp.concatenate(
            [a_ref[...][8:, K:], apeek_ref[...][:, K:]], axis=0)
        fc_ref[...] = jnp.concatenate([fcleft_win, a_win], axis=1)
        fw_ref[...] = (c1024 == K * i + r1024 + 8).astype(f32)


def kernel(A_dense, O, rot_rows, rot_cols, wav_idx, act_idx):
    f32 = jnp.float32
    sds = jax.ShapeDtypeStruct

    strips = pl.pallas_call(
        _strip_kernel,
        grid=(1,),
        in_specs=[
            pl.BlockSpec((K, N), lambda i: (0, 0)),
            pl.BlockSpec((N, K), lambda i: (0, 0)),
            pl.BlockSpec((8, 16, 16), lambda i: (0, 0, 0)),
        ],
        out_specs=[
            pl.BlockSpec((K, K), lambda i: (0, 0)),
            pl.BlockSpec((K, N), lambda i: (0, 0)),
            pl.BlockSpec((K, N), lambda i: (0, 0)),
            pl.BlockSpec((NB, K), lambda i: (0, 0)),
            pl.BlockSpec((NB, K), lambda i: (0, 0)),
            pl.BlockSpec((KA, KA), lambda i: (0, 0)),
            pl.BlockSpec((KA, NB), lambda i: (0, 0)),
            pl.BlockSpec((NB, KA), lambda i: (0, 0)),
            pl.BlockSpec((KA, K), lambda i: (0, 0)),
            pl.BlockSpec((8, 8), lambda i: (0, 0)),
            pl.BlockSpec((8, N), lambda i: (0, 0)),
        ],
        out_shape=[
            sds((K, K), f32),
            sds((K, N), f32),
            sds((K, N), f32),
            sds((NB, K), f32),
            sds((NB, K), f32),
            sds((KA, KA), f32),
            sds((KA, NB), f32),
            sds((NB, KA), f32),
            sds((KA, K), f32),
            sds((8, 8), f32),
            sds((8, N), f32),
        ],
    )(A_dense, A_dense, O)
    (b, d_top, ar_top, d_left, ar_left,
     fctl, fctr, fcleft, fwtl, mc, mw) = strips

    d, a_rec, right, fc, fw = pl.pallas_call(
        _assemble_kernel,
        grid=(8,),
        in_specs=[
            pl.BlockSpec((K, N), lambda i: (i, 0)),
            pl.BlockSpec((8, N),
                         lambda i: (jnp.minimum(16 * (i + 1), 127), 0)),
            pl.BlockSpec((K, K), lambda i: (0, 0)),
            pl.BlockSpec((K, N), lambda i: (0, 0)),
            pl.BlockSpec((K, N), lambda i: (0, 0)),
            pl.BlockSpec((K, K), lambda i: (jnp.maximum(i - 1, 0), 0)),
            pl.BlockSpec((K, K), lambda i: (jnp.maximum(i - 1, 0), 0)),
            pl.BlockSpec((KA, KA), lambda i: (0, 0)),
            pl.BlockSpec((KA, NB), lambda i: (0, 0)),
            pl.BlockSpec((K, KA), lambda i: (jnp.maximum(i - 1, 0), 0)),
            pl.BlockSpec((8, KA),
                         lambda i: (jnp.minimum(16 * i, 111), 0)),
            pl.BlockSpec((KA, K), lambda i: (0, 0)),
        ],
        out_specs=[
            pl.BlockSpec((K, N), lambda i: (i, 0)),
            pl.BlockSpec((K, N), lambda i: (i, 0)),
            pl.BlockSpec((K, N), lambda i: (i, 0)),
            pl.BlockSpec((K, NA), lambda i: (i, 0)),
            pl.BlockSpec((K, N), lambda i: (i, 0)),
        ],
        out_shape=[
            sds((N, N), f32), sds((N, N), f32), sds((N, N), f32),
            sds((NA, NA), f32), sds((NA, N), f32),
        ],
    )(A_dense, A_dense, b, d_top, ar_top, d_left, ar_left,
      fctl, fctr, fcleft, fcleft, fwtl)

    return (a_rec, right, d, mc, fc, mw, fw)
